# baseline (speedup 1.0000x reference)
import functools
import jax
import jax.numpy as jnp
from jax import lax
from jax.experimental import pallas as pl
from jax.experimental.pallas import tpu as pltpu
from jax.experimental.pallas import tpu_sc as plsc

f32 = jnp.float32
i32 = jnp.int32

N = 10000
D = 128
HD = 64
GAMMA = 0.1
E = 320000
EK = 200000

NC = 2
NS = 16
NW = NC * NS
CH = 128

NPAD = 10240
NPD = 10112
RPD = NPD // NS
BLK = 1024
GRID = NPAD // BLK

NB = 2
Q0M, Q1M = 144, 16
Q0K, Q1K = 96, 16
TCH_M = NS * (Q0M + Q1M)
TCH_K = NS * (Q0K + Q1K)
EP_M = TCH_M * CH
EP_K = TCH_K * CH
D0M, D1M = 112, 48
D0K, D1K = 80, 32

_mesh = plsc.VectorSubcoreMesh(core_axis_name="c", subcore_axis_name="s")


@functools.partial(
    pl.kernel,
    out_type=jax.ShapeDtypeStruct((2, 4, NPAD, 16), f32),
    mesh=_mesh,
    scratch_types=[
        pltpu.VMEM((D0M, CH), i32),
        pltpu.VMEM((CH, 16), f32),
        pltpu.VMEM((CH, 16), f32),
        pltpu.VMEM_SHARED((NPD, 16), f32),
        pltpu.SemaphoreType.DMA,
    ],
    compiler_params=pltpu.CompilerParams(use_tc_tiling_on_sc=False),
)
def _deg_kernel(srcm, dstm, srck, dstk, out,
                sia, ones_v, zb, acc, dsem):
    cid = lax.axis_index("c")
    sid = lax.axis_index("s")
    row0 = sid * RPD

    def _fill(r, _):
        ones_v[r, pl.ds(0, 16)] = jnp.ones((16,), f32)
        zb[r, pl.ds(0, 16)] = jnp.zeros((16,), f32)
        return 0
    lax.fori_loop(0, CH, _fill, 0)

    def _scatter(arr, q0, q1):
        def _run(nch, base):
            pltpu.sync_copy(arr.at[pl.ds(base, nch)], sia.at[pl.ds(0, nch)])

            def _grp(t, _):
                for b in range(8):
                    pltpu.async_copy(ones_v, acc.at[sia.at[t * 8 + b]],
                                     dsem, add=True)
                for b in range(8):
                    pltpu.make_async_copy(ones_v, acc.at[sia.at[0]],
                                          dsem).wait()
                return 0
            lax.fori_loop(0, nch // 8, _grp, 0)

        @pl.when(cid == 0)
        def _c0():
            _run(q0, sid * q0)

        @pl.when(cid == 1)
        def _c1():
            _run(q1, NS * q0 + sid * q1)

    for p, (arr, q0, q1) in enumerate(((srcm, D0M, D1M), (dstm, D0M, D1M),
                                       (srck, D0K, D1K), (dstk, D0K, D1K))):
        for part in range(RPD // CH):
            pltpu.sync_copy(zb, acc.at[pl.ds(row0 + part * CH, CH)])
        rem = RPD - (RPD // CH) * CH
        pltpu.sync_copy(zb.at[pl.ds(0, rem)],
                        acc.at[pl.ds(row0 + (RPD // CH) * CH, rem)])
        plsc.subcore_barrier()
        _scatter(arr, q0, q1)
        plsc.subcore_barrier()
        pltpu.sync_copy(acc.at[pl.ds(row0, RPD)],
                        out.at[cid, p, pl.ds(row0, RPD)])

        @pl.when(sid == NS - 1)
        def _tail():
            pltpu.sync_copy(zb.at[pl.ds(0, NPAD - NPD)],
                            out.at[cid, p, pl.ds(NPD, NPAD - NPD)])


@functools.partial(
    pl.kernel,
    out_type=(jax.ShapeDtypeStruct((2, NPAD, HD), f32),
              jax.ShapeDtypeStruct((2, NPAD, HD), f32)),
    mesh=_mesh,
    scratch_types=[
        pltpu.VMEM((2 * NB, CH, HD), f32),
        pltpu.VMEM((Q0M, CH), i32),
        pltpu.VMEM((Q0M, CH), i32),
        pltpu.VMEM_SHARED((NPD, HD), f32),
        pltpu.SemaphoreType.DMA,
        pltpu.SemaphoreType.DMA,
        pltpu.SemaphoreType.DMA,
        pltpu.SemaphoreType.DMA,
    ],
    compiler_params=pltpu.CompilerParams(use_tc_tiling_on_sc=False),
)
def _agg_kernel(hsm, hsk, srcm, dstm, srck, dstk, outm, outk,
                rows, sia, dia, acc, g0, g1, s0, s1):
    cid = lax.axis_index("c")
    sid = lax.axis_index("s")
    row0 = sid * RPD
    gsem = (g0, g1)
    ssem = (s0, s1)
    zb = rows.at[2 * NB - 1]

    def _zero_zb():
        def _zfill(r, _):
            for q in range(HD // 16):
                zb[r, pl.ds(q * 16, 16)] = jnp.zeros((16,), f32)
            return 0
        lax.fori_loop(0, CH, _zfill, 0)

    def _zero_acc():
        for part in range(RPD // CH):
            pltpu.sync_copy(zb, acc.at[pl.ds(row0 + part * CH, CH)])
        rem = RPD - (RPD // CH) * CH
        pltpu.sync_copy(zb.at[pl.ds(0, rem)],
                        acc.at[pl.ds(row0 + (RPD // CH) * CH, rem)])

    _zero_zb()
    _zero_acc()
    plsc.subcore_barrier()

    def _pipeline(tab, s2, d2, nch, base):
        ng = nch // NB
        pltpu.sync_copy(s2.at[pl.ds(base, nch)], sia.at[pl.ds(0, nch)])
        pltpu.sync_copy(d2.at[pl.ds(base, nch)], dia.at[pl.ds(0, nch)])

        def fire_g(bank, g):
            for b in range(NB):
                pltpu.async_copy(tab.at[sia.at[g * NB + b]],
                                 rows.at[bank * NB + b], gsem[bank])

        def drain_g(bank):
            for b in range(NB):
                pltpu.make_async_copy(tab.at[sia.at[0]],
                                      rows.at[bank * NB + b],
                                      gsem[bank]).wait()

        def fire_s(bank, g):
            for b in range(NB):
                pltpu.async_copy(rows.at[bank * NB + b],
                                 acc.at[dia.at[g * NB + b]], ssem[bank],
                                 add=True)

        def drain_s(bank):
            for b in range(NB):
                pltpu.make_async_copy(rows.at[bank * NB + b],
                                      acc.at[dia.at[0]], ssem[bank]).wait()

        fire_g(0, 0)

        def _pair(t, _):
            ga = 2 * t
            fire_g(1, ga + 1)
            drain_g(0)
            fire_s(0, ga)
            drain_s(0)
            fire_g(0, lax.rem(ga + 2, ng))
            drain_g(1)
            fire_s(1, ga + 1)
            drain_s(1)
            return 0
        lax.fori_loop(0, ng // 2, _pair, 0)
        drain_g(0)

    for tab, s2, d2, outx, q0, q1 in (
            (hsm, srcm, dstm, outm, Q0M, Q1M),
            (hsk, srck, dstk, outk, Q0K, Q1K)):

        @pl.when(cid == 0)
        def _c0():
            _pipeline(tab, s2, d2, q0, sid * q0)

        @pl.when(cid == 1)
        def _c1():
            _pipeline(tab, s2, d2, q1, NS * q0 + sid * q1)

        plsc.subcore_barrier()
        pltpu.sync_copy(acc.at[pl.ds(row0, RPD)],
                        outx.at[cid, pl.ds(row0, RPD)])
        _zero_zb()

        @pl.when(sid == NS - 1)
        def _tail():
            pltpu.sync_copy(zb.at[pl.ds(0, NPAD - NPD)],
                            outx.at[cid, pl.ds(NPD, NPAD - NPD)])
        if tab is hsm:
            _zero_acc()
            plsc.subcore_barrier()


def _invs_body(d_ref, iom_ref, iim_ref, iok_ref, iik_ref, swm_ref):
    d = d_ref[...]
    dm_o = (d[0, 0] + d[1, 0])[:, 0:1]
    dm_i = (d[0, 1] + d[1, 1])[:, 0:1]
    dk_o = (d[0, 2] + d[1, 2])[:, 0:1]
    dk_i = (d[0, 3] + d[1, 3])[:, 0:1]
    iom = lax.rsqrt(dm_o + 1.0)
    iim = lax.rsqrt(dm_i + 1.0)
    iok_ref[...] = jnp.where(dk_o > 0, lax.rsqrt(jnp.maximum(dk_o, 1.0)), 0.0)
    iik_ref[...] = jnp.where(dk_i > 0, lax.rsqrt(jnp.maximum(dk_i, 1.0)), 0.0)
    iom_ref[...] = iom
    iim_ref[...] = iim
    swm_ref[...] = iom * iim


def _invs(deg):
    shp = jax.ShapeDtypeStruct((NPAD, 1), f32)
    return pl.pallas_call(
        _invs_body,
        grid=(GRID,),
        in_specs=[pl.BlockSpec((2, 4, BLK, 16), lambda i: (0, 0, i, 0))],
        out_specs=[pl.BlockSpec((BLK, 1), lambda i: (i, 0))] * 5,
        out_shape=(shp,) * 5,
    )(deg)


def _kA_body(x_ref, w_ref, sr_ref, dr_ref, b_ref, db_ref, iom_ref, iok_ref,
             h_ref, hsm_ref, hsk_ref, s_ref, dk_ref):
    x = x_ref[...]
    h = jnp.dot(x, w_ref[...], preferred_element_type=f32)
    s = jax.nn.sigmoid(jnp.dot(x, sr_ref[...], preferred_element_type=f32)
                       + b_ref[...])
    dk = jnp.dot(x, dr_ref[...], preferred_element_type=f32) + db_ref[...]
    h_ref[...] = h
    hsm_ref[...] = h * iom_ref[...]
    hsk_ref[...] = h * iok_ref[...]
    s_ref[...] = s
    dk_ref[...] = dk


def _mm_specs(din):
    full = lambda shape: pl.BlockSpec(shape, lambda i: (0,) * len(shape))
    return [
        pl.BlockSpec((BLK, din), lambda i: (i, 0)),
        full((din, HD)),
        full((din, HD)),
        full((din, HD)),
        full((1, HD)),
        full((1, HD)),
        pl.BlockSpec((BLK, 1), lambda i: (i, 0)),
        pl.BlockSpec((BLK, 1), lambda i: (i, 0)),
    ]


_ROWOUT = [pl.BlockSpec((BLK, HD), lambda i: (i, 0))] * 5
_SHP5 = (jax.ShapeDtypeStruct((NPAD, HD), f32),) * 5


def _kA(x, w, sr, dr, b, db, iom, iok):
    return pl.pallas_call(
        _kA_body,
        grid=(GRID,),
        in_specs=_mm_specs(x.shape[1]),
        out_specs=_ROWOUT,
        out_shape=_SHP5,
    )(x, w, sr, dr, b, db, iom, iok)


def _combine(am_ref, ak_ref, h_ref, s_ref, dk_ref, iim_ref, iik_ref, swm_ref):
    am = am_ref[0] + am_ref[1]
    ak = ak_ref[0] + ak_ref[1]
    h = h_ref[...]
    s = s_ref[...]
    h_main = iim_ref[...] * am + swm_ref[...] * h
    tmp_knn = iik_ref[...] * ak
    return s * h_main + (1.0 - s) * tmp_knn + GAMMA * dk_ref[...] * h


def _kBA_body(am_ref, ak_ref, h_ref, s_ref, dk_ref, iim_ref, iik_ref, swm_ref,
              w_ref, sr_ref, dr_ref, b_ref, db_ref, iom_ref, iok_ref,
              h2_ref, hsm_ref, hsk_ref, s2_ref, dk2_ref):
    x2 = _combine(am_ref, ak_ref, h_ref, s_ref, dk_ref,
                  iim_ref, iik_ref, swm_ref)
    h2 = jnp.dot(x2, w_ref[...], preferred_element_type=f32)
    s2 = jax.nn.sigmoid(jnp.dot(x2, sr_ref[...], preferred_element_type=f32)
                        + b_ref[...])
    dk2 = jnp.dot(x2, dr_ref[...], preferred_element_type=f32) + db_ref[...]
    h2_ref[...] = h2
    hsm_ref[...] = h2 * iom_ref[...]
    hsk_ref[...] = h2 * iok_ref[...]
    s2_ref[...] = s2
    dk2_ref[...] = dk2


def _comb_specs():
    return [
        pl.BlockSpec((2, BLK, HD), lambda i: (0, i, 0)),
        pl.BlockSpec((2, BLK, HD), lambda i: (0, i, 0)),
        pl.BlockSpec((BLK, HD), lambda i: (i, 0)),
        pl.BlockSpec((BLK, HD), lambda i: (i, 0)),
        pl.BlockSpec((BLK, HD), lambda i: (i, 0)),
        pl.BlockSpec((BLK, 1), lambda i: (i, 0)),
        pl.BlockSpec((BLK, 1), lambda i: (i, 0)),
        pl.BlockSpec((BLK, 1), lambda i: (i, 0)),
    ]


def _kBA(am, ak, h, s, dk, iim, iik, swm, w, sr, dr, b, db, iom, iok):
    full = lambda shape: pl.BlockSpec(shape, lambda i: (0,) * len(shape))
    in_specs = _comb_specs() + [
        full((HD, HD)), full((HD, HD)), full((HD, HD)),
        full((1, HD)), full((1, HD)),
        pl.BlockSpec((BLK, 1), lambda i: (i, 0)),
        pl.BlockSpec((BLK, 1), lambda i: (i, 0)),
    ]
    return pl.pallas_call(
        _kBA_body,
        grid=(GRID,),
        in_specs=in_specs,
        out_specs=_ROWOUT,
        out_shape=_SHP5,
    )(am, ak, h, s, dk, iim, iik, swm, w, sr, dr, b, db, iom, iok)


def _kB_body(am_ref, ak_ref, h_ref, s_ref, dk_ref, iim_ref, iik_ref, swm_ref,
             out_ref):
    out_ref[...] = _combine(am_ref, ak_ref, h_ref, s_ref, dk_ref,
                            iim_ref, iik_ref, swm_ref)


def _kB(am, ak, h, s, dk, iim, iik, swm):
    return pl.pallas_call(
        _kB_body,
        grid=(GRID,),
        in_specs=_comb_specs(),
        out_specs=pl.BlockSpec((BLK, HD), lambda i: (i, 0)),
        out_shape=jax.ShapeDtypeStruct((NPAD, HD), f32),
    )(am, ak, h, s, dk, iim, iik, swm)


def _pad_edges(idx, ep):
    idxp = jnp.concatenate([idx, jnp.full((ep - idx.shape[0],), N, i32)])
    return idxp.reshape(ep // CH, CH)


def kernel(feat, edge_index, knn_edge_index, W0, W1, scores0, scores1,
           bias0, bias1, Dk0, Dk1, Dbias0, Dbias1):
    featp = jnp.pad(feat, ((0, NPAD - N), (0, 0)))
    srcm = _pad_edges(edge_index[0], EP_M)
    dstm = _pad_edges(edge_index[1], EP_M)
    srck = _pad_edges(knn_edge_index[0], EP_K)
    dstk = _pad_edges(knn_edge_index[1], EP_K)

    deg = _deg_kernel(srcm, dstm, srck, dstk)
    iom, iim, iok, iik, swm = _invs(deg)

    sr0 = jnp.broadcast_to(scores0, (D, HD))
    dr0 = jnp.broadcast_to(Dk0, (D, HD))
    sr1 = jnp.broadcast_to(scores1, (HD, HD))
    dr1 = jnp.broadcast_to(Dk1, (HD, HD))
    b0 = jnp.broadcast_to(bias0.reshape(1, 1), (1, HD))
    db0 = jnp.broadcast_to(Dbias0.reshape(1, 1), (1, HD))
    b1 = jnp.broadcast_to(bias1.reshape(1, 1), (1, HD))
    db1 = jnp.broadcast_to(Dbias1.reshape(1, 1), (1, HD))

    h1, hs1m, hs1k, s1, dk1 = _kA(featp, W0, sr0, dr0, b0, db0, iom, iok)
    am1, ak1 = _agg_kernel(hs1m, hs1k, srcm, dstm, srck, dstk)
    h2, hs2m, hs2k, s2, dk2 = _kBA(am1, ak1, h1, s1, dk1, iim, iik, swm,
                                   W1, sr1, dr1, b1, db1, iom, iok)
    am2, ak2 = _agg_kernel(hs2m, hs2k, srcm, dstm, srck, dstk)
    x3 = _kB(am2, ak2, h2, s2, dk2, iim, iik, swm)
    return x3[:N]


# speedup vs baseline: 1.1606x; 1.1606x over previous
import functools
import jax
import jax.numpy as jnp
from jax import lax
from jax.experimental import pallas as pl
from jax.experimental.pallas import tpu as pltpu
from jax.experimental.pallas import tpu_sc as plsc

f32 = jnp.float32
i32 = jnp.int32

N = 10000
D = 128
HD = 64
GAMMA = 0.1
E = 320000
EK = 200000

NC = 2
NS = 16
NW = NC * NS
CH = 128

NPAD = 10240
NPD = 10112
RPD = NPD // NS
BLK = 1024
GRID = NPAD // BLK

NB = 2
Q0M, Q1M = 160, 0
Q0K, Q1K = 112, 0
TCH_M = NS * (Q0M + Q1M)
TCH_K = NS * (Q0K + Q1K)
EP_M = TCH_M * CH
EP_K = TCH_K * CH
D0M, D1M = 112, 48
D0K, D1K = 80, 32

_mesh = plsc.VectorSubcoreMesh(core_axis_name="c", subcore_axis_name="s")


@functools.partial(
    pl.kernel,
    out_type=jax.ShapeDtypeStruct((2, 4, NPAD, 16), f32),
    mesh=_mesh,
    scratch_types=[
        pltpu.VMEM((D0M, CH), i32),
        pltpu.VMEM((CH, 16), f32),
        pltpu.VMEM((CH, 16), f32),
        pltpu.VMEM_SHARED((NPD, 16), f32),
        pltpu.SemaphoreType.DMA,
    ],
    compiler_params=pltpu.CompilerParams(use_tc_tiling_on_sc=False),
)
def _deg_kernel(srcm, dstm, srck, dstk, out,
                sia, ones_v, zb, acc, dsem):
    cid = lax.axis_index("c")
    sid = lax.axis_index("s")
    row0 = sid * RPD

    def _fill(r, _):
        ones_v[r, pl.ds(0, 16)] = jnp.ones((16,), f32)
        zb[r, pl.ds(0, 16)] = jnp.zeros((16,), f32)
        return 0
    lax.fori_loop(0, CH, _fill, 0)

    def _scatter(arr, q0, q1):
        def _run(nch, base):
            pltpu.sync_copy(arr.at[pl.ds(base, nch)], sia.at[pl.ds(0, nch)])

            def _grp(t, _):
                for b in range(8):
                    pltpu.async_copy(ones_v, acc.at[sia.at[t * 8 + b]],
                                     dsem, add=True)
                for b in range(8):
                    pltpu.make_async_copy(ones_v, acc.at[sia.at[0]],
                                          dsem).wait()
                return 0
            lax.fori_loop(0, nch // 8, _grp, 0)

        @pl.when(cid == 0)
        def _c0():
            _run(q0, sid * q0)

        @pl.when(cid == 1)
        def _c1():
            _run(q1, NS * q0 + sid * q1)

    for p, (arr, q0, q1) in enumerate(((srcm, D0M, D1M), (dstm, D0M, D1M),
                                       (srck, D0K, D1K), (dstk, D0K, D1K))):
        for part in range(RPD // CH):
            pltpu.sync_copy(zb, acc.at[pl.ds(row0 + part * CH, CH)])
        rem = RPD - (RPD // CH) * CH
        pltpu.sync_copy(zb.at[pl.ds(0, rem)],
                        acc.at[pl.ds(row0 + (RPD // CH) * CH, rem)])
        plsc.subcore_barrier()
        _scatter(arr, q0, q1)
        plsc.subcore_barrier()
        pltpu.sync_copy(acc.at[pl.ds(row0, RPD)],
                        out.at[cid, p, pl.ds(row0, RPD)])

        @pl.when(sid == NS - 1)
        def _tail():
            pltpu.sync_copy(zb.at[pl.ds(0, NPAD - NPD)],
                            out.at[cid, p, pl.ds(NPD, NPAD - NPD)])


NCHT_M = TCH_M // NW
NCHT_K = TCH_K // NW


@functools.partial(
    pl.kernel,
    out_type=(jax.ShapeDtypeStruct((2, NPAD, HD), f32),
              jax.ShapeDtypeStruct((2, NPAD, HD), f32)),
    mesh=_mesh,
    scratch_types=[
        pltpu.VMEM((CH, HD), f32),
        pltpu.VMEM((CH, HD), f32),
        pltpu.VMEM((CH,), i32),
        pltpu.VMEM((CH,), i32),
        pltpu.VMEM((CH,), i32),
        pltpu.VMEM((CH,), i32),
        pltpu.VMEM((CH, HD), f32),
        pltpu.VMEM_SHARED((NPD, HD), f32),
        pltpu.VMEM_SHARED((NPD, HD), f32),
        pltpu.SemaphoreType.DMA,
        pltpu.SemaphoreType.DMA,
        pltpu.SemaphoreType.DMA,
        pltpu.SemaphoreType.DMA,
        pltpu.SemaphoreType.DMA,
        pltpu.SemaphoreType.DMA,
    ],
    compiler_params=pltpu.CompilerParams(use_tc_tiling_on_sc=False),
)
def _agg_kernel(hsm, hsk, srcm, dstm, srck, dstk, outm, outk,
                rowsa, rowsb, sia, dia, sib, dib, zb, accm, acck,
                ia, ib, ga, gb, sa_, sb_):
    cid = lax.axis_index("c")
    sid = lax.axis_index("s")
    wid = cid * NS + sid
    row0 = sid * RPD

    def _zfill(r, _):
        for q in range(HD // 16):
            zb[r, pl.ds(q * 16, 16)] = jnp.zeros((16,), f32)
        return 0
    lax.fori_loop(0, CH, _zfill, 0)
    for acc in (accm, acck):
        for part in range(RPD // CH):
            pltpu.sync_copy(zb, acc.at[pl.ds(row0 + part * CH, CH)])
        rem = RPD - (RPD // CH) * CH
        pltpu.sync_copy(zb.at[pl.ds(0, rem)],
                        acc.at[pl.ds(row0 + (RPD // CH) * CH, rem)])
    plsc.subcore_barrier()

    for tab, s2, d2, acc, ncht in (
            (hsm, srcm, dstm, accm, NCHT_M),
            (hsk, srck, dstk, acck, NCHT_K)):
        base = wid * ncht

        def _pair(t, _):
            r = base + 2 * t
            da1 = pltpu.async_copy(s2.at[r], sia, ia)
            da2 = pltpu.async_copy(d2.at[r], dia, ia)
            db1 = pltpu.async_copy(s2.at[r + 1], sib, ib)
            db2 = pltpu.async_copy(d2.at[r + 1], dib, ib)
            da1.wait()
            da2.wait()
            gda = pltpu.async_copy(tab.at[sia], rowsa, ga)
            db1.wait()
            db2.wait()
            gdb = pltpu.async_copy(tab.at[sib], rowsb, gb)
            gda.wait()
            sda = pltpu.async_copy(rowsa, acc.at[dia], sa_, add=True)
            gdb.wait()
            sdb = pltpu.async_copy(rowsb, acc.at[dib], sb_, add=True)
            sda.wait()
            sdb.wait()
            return 0
        lax.fori_loop(0, ncht // 2, _pair, 0)
    plsc.subcore_barrier()

    pltpu.sync_copy(accm.at[pl.ds(row0, RPD)], outm.at[cid, pl.ds(row0, RPD)])
    pltpu.sync_copy(acck.at[pl.ds(row0, RPD)], outk.at[cid, pl.ds(row0, RPD)])

    @pl.when(sid == NS - 1)
    def _tail():
        pltpu.sync_copy(zb.at[pl.ds(0, NPAD - NPD)],
                        outm.at[cid, pl.ds(NPD, NPAD - NPD)])
        pltpu.sync_copy(zb.at[pl.ds(0, NPAD - NPD)],
                        outk.at[cid, pl.ds(NPD, NPAD - NPD)])


def _invs_body(d_ref, iom_ref, iim_ref, iok_ref, iik_ref, swm_ref):
    d = d_ref[...]
    dm_o = (d[0, 0] + d[1, 0])[:, 0:1]
    dm_i = (d[0, 1] + d[1, 1])[:, 0:1]
    dk_o = (d[0, 2] + d[1, 2])[:, 0:1]
    dk_i = (d[0, 3] + d[1, 3])[:, 0:1]
    iom = lax.rsqrt(dm_o + 1.0)
    iim = lax.rsqrt(dm_i + 1.0)
    iok_ref[...] = jnp.where(dk_o > 0, lax.rsqrt(jnp.maximum(dk_o, 1.0)), 0.0)
    iik_ref[...] = jnp.where(dk_i > 0, lax.rsqrt(jnp.maximum(dk_i, 1.0)), 0.0)
    iom_ref[...] = iom
    iim_ref[...] = iim
    swm_ref[...] = iom * iim


def _invs(deg):
    shp = jax.ShapeDtypeStruct((NPAD, 1), f32)
    return pl.pallas_call(
        _invs_body,
        grid=(GRID,),
        in_specs=[pl.BlockSpec((2, 4, BLK, 16), lambda i: (0, 0, i, 0))],
        out_specs=[pl.BlockSpec((BLK, 1), lambda i: (i, 0))] * 5,
        out_shape=(shp,) * 5,
    )(deg)


def _kA_body(x_ref, w_ref, sr_ref, dr_ref, b_ref, db_ref, iom_ref, iok_ref,
             h_ref, hsm_ref, hsk_ref, s_ref, dk_ref):
    x = x_ref[...]
    h = jnp.dot(x, w_ref[...], preferred_element_type=f32)
    s = jax.nn.sigmoid(jnp.dot(x, sr_ref[...], preferred_element_type=f32)
                       + b_ref[...])
    dk = jnp.dot(x, dr_ref[...], preferred_element_type=f32) + db_ref[...]
    h_ref[...] = h
    hsm_ref[...] = h * iom_ref[...]
    hsk_ref[...] = h * iok_ref[...]
    s_ref[...] = s
    dk_ref[...] = dk


def _mm_specs(din):
    full = lambda shape: pl.BlockSpec(shape, lambda i: (0,) * len(shape))
    return [
        pl.BlockSpec((BLK, din), lambda i: (i, 0)),
        full((din, HD)),
        full((din, HD)),
        full((din, HD)),
        full((1, HD)),
        full((1, HD)),
        pl.BlockSpec((BLK, 1), lambda i: (i, 0)),
        pl.BlockSpec((BLK, 1), lambda i: (i, 0)),
    ]


_ROWOUT = [pl.BlockSpec((BLK, HD), lambda i: (i, 0))] * 5
_SHP5 = (jax.ShapeDtypeStruct((NPAD, HD), f32),) * 5


def _kA(x, w, sr, dr, b, db, iom, iok):
    return pl.pallas_call(
        _kA_body,
        grid=(GRID,),
        in_specs=_mm_specs(x.shape[1]),
        out_specs=_ROWOUT,
        out_shape=_SHP5,
    )(x, w, sr, dr, b, db, iom, iok)


def _combine(am_ref, ak_ref, h_ref, s_ref, dk_ref, iim_ref, iik_ref, swm_ref):
    am = am_ref[0] + am_ref[1]
    ak = ak_ref[0] + ak_ref[1]
    h = h_ref[...]
    s = s_ref[...]
    h_main = iim_ref[...] * am + swm_ref[...] * h
    tmp_knn = iik_ref[...] * ak
    return s * h_main + (1.0 - s) * tmp_knn + GAMMA * dk_ref[...] * h


def _kBA_body(am_ref, ak_ref, h_ref, s_ref, dk_ref, iim_ref, iik_ref, swm_ref,
              w_ref, sr_ref, dr_ref, b_ref, db_ref, iom_ref, iok_ref,
              h2_ref, hsm_ref, hsk_ref, s2_ref, dk2_ref):
    x2 = _combine(am_ref, ak_ref, h_ref, s_ref, dk_ref,
                  iim_ref, iik_ref, swm_ref)
    h2 = jnp.dot(x2, w_ref[...], preferred_element_type=f32)
    s2 = jax.nn.sigmoid(jnp.dot(x2, sr_ref[...], preferred_element_type=f32)
                        + b_ref[...])
    dk2 = jnp.dot(x2, dr_ref[...], preferred_element_type=f32) + db_ref[...]
    h2_ref[...] = h2
    hsm_ref[...] = h2 * iom_ref[...]
    hsk_ref[...] = h2 * iok_ref[...]
    s2_ref[...] = s2
    dk2_ref[...] = dk2


def _comb_specs():
    return [
        pl.BlockSpec((2, BLK, HD), lambda i: (0, i, 0)),
        pl.BlockSpec((2, BLK, HD), lambda i: (0, i, 0)),
        pl.BlockSpec((BLK, HD), lambda i: (i, 0)),
        pl.BlockSpec((BLK, HD), lambda i: (i, 0)),
        pl.BlockSpec((BLK, HD), lambda i: (i, 0)),
        pl.BlockSpec((BLK, 1), lambda i: (i, 0)),
        pl.BlockSpec((BLK, 1), lambda i: (i, 0)),
        pl.BlockSpec((BLK, 1), lambda i: (i, 0)),
    ]


def _kBA(am, ak, h, s, dk, iim, iik, swm, w, sr, dr, b, db, iom, iok):
    full = lambda shape: pl.BlockSpec(shape, lambda i: (0,) * len(shape))
    in_specs = _comb_specs() + [
        full((HD, HD)), full((HD, HD)), full((HD, HD)),
        full((1, HD)), full((1, HD)),
        pl.BlockSpec((BLK, 1), lambda i: (i, 0)),
        pl.BlockSpec((BLK, 1), lambda i: (i, 0)),
    ]
    return pl.pallas_call(
        _kBA_body,
        grid=(GRID,),
        in_specs=in_specs,
        out_specs=_ROWOUT,
        out_shape=_SHP5,
    )(am, ak, h, s, dk, iim, iik, swm, w, sr, dr, b, db, iom, iok)


def _kB_body(am_ref, ak_ref, h_ref, s_ref, dk_ref, iim_ref, iik_ref, swm_ref,
             out_ref):
    out_ref[...] = _combine(am_ref, ak_ref, h_ref, s_ref, dk_ref,
                            iim_ref, iik_ref, swm_ref)


def _kB(am, ak, h, s, dk, iim, iik, swm):
    return pl.pallas_call(
        _kB_body,
        grid=(GRID,),
        in_specs=_comb_specs(),
        out_specs=pl.BlockSpec((BLK, HD), lambda i: (i, 0)),
        out_shape=jax.ShapeDtypeStruct((NPAD, HD), f32),
    )(am, ak, h, s, dk, iim, iik, swm)


def _pad_edges(idx, ep):
    idxp = jnp.concatenate([idx, jnp.full((ep - idx.shape[0],), N, i32)])
    return idxp.reshape(ep // CH, CH)


def kernel(feat, edge_index, knn_edge_index, W0, W1, scores0, scores1,
           bias0, bias1, Dk0, Dk1, Dbias0, Dbias1):
    featp = jnp.pad(feat, ((0, NPAD - N), (0, 0)))
    srcm = _pad_edges(edge_index[0], EP_M)
    dstm = _pad_edges(edge_index[1], EP_M)
    srck = _pad_edges(knn_edge_index[0], EP_K)
    dstk = _pad_edges(knn_edge_index[1], EP_K)

    deg = _deg_kernel(srcm, dstm, srck, dstk)
    iom, iim, iok, iik, swm = _invs(deg)

    sr0 = jnp.broadcast_to(scores0, (D, HD))
    dr0 = jnp.broadcast_to(Dk0, (D, HD))
    sr1 = jnp.broadcast_to(scores1, (HD, HD))
    dr1 = jnp.broadcast_to(Dk1, (HD, HD))
    b0 = jnp.broadcast_to(bias0.reshape(1, 1), (1, HD))
    db0 = jnp.broadcast_to(Dbias0.reshape(1, 1), (1, HD))
    b1 = jnp.broadcast_to(bias1.reshape(1, 1), (1, HD))
    db1 = jnp.broadcast_to(Dbias1.reshape(1, 1), (1, HD))

    h1, hs1m, hs1k, s1, dk1 = _kA(featp, W0, sr0, dr0, b0, db0, iom, iok)
    am1, ak1 = _agg_kernel(hs1m, hs1k, srcm, dstm, srck, dstk)
    h2, hs2m, hs2k, s2, dk2 = _kBA(am1, ak1, h1, s1, dk1, iim, iik, swm,
                                   W1, sr1, dr1, b1, db1, iom, iok)
    am2, ak2 = _agg_kernel(hs2m, hs2k, srcm, dstm, srck, dstk)
    x3 = _kB(am2, ak2, h2, s2, dk2, iim, iik, swm)
    return x3[:N]


# speedup vs baseline: 1.8003x; 1.5511x over previous
import functools
import jax
import jax.numpy as jnp
from jax import lax
from jax.experimental import pallas as pl
from jax.experimental.pallas import tpu as pltpu
from jax.experimental.pallas import tpu_sc as plsc

f32 = jnp.float32
i32 = jnp.int32

N = 10000
D = 128
HD = 64
GAMMA = 0.1
E = 320000
EK = 200000

NC = 2
NS = 16
NW = NC * NS
CH = 128

NPAD = 10240
NPD = 10112
RPD = NPD // NS
BLK = 1024
GRID = NPAD // BLK

NCH_M = 79
NCH_K = 49
EPT_M = NCH_M * CH
EPT_K = NCH_K * CH
EP_M = EPT_M * NW
EP_K = EPT_K * NW

_mesh = plsc.VectorSubcoreMesh(core_axis_name="c", subcore_axis_name="s")


@functools.partial(
    pl.kernel,
    out_type=jax.ShapeDtypeStruct((2, 4, NPAD, 16), f32),
    mesh=_mesh,
    scratch_types=[
        pltpu.VMEM((CH,), i32),
        pltpu.VMEM((CH, 16), f32),
        pltpu.VMEM((RPD, 16), f32),
        pltpu.VMEM_SHARED((NPD, 16), f32),
        pltpu.VMEM_SHARED((NPD, 16), f32),
    ],
    compiler_params=pltpu.CompilerParams(use_tc_tiling_on_sc=False),
)
def _deg_kernel(srcm, dstm, srck, dstk, out,
                ib, ones_v, zb, a0, a1):
    cid = lax.axis_index("c")
    sid = lax.axis_index("s")
    wid = cid * NS + sid
    row0 = sid * RPD

    def _fill(r, _):
        ones_v[r, pl.ds(0, 16)] = jnp.ones((16,), f32)
        return 0
    lax.fori_loop(0, CH, _fill, 0)

    def _zfill(r, _):
        zb[r, pl.ds(0, 16)] = jnp.zeros((16,), f32)
        return 0
    lax.fori_loop(0, RPD, _zfill, 0)

    for p, (arrs, nch, ept) in enumerate(
            (((srcm, dstm), NCH_M, EPT_M), ((srck, dstk), NCH_K, EPT_K))):
        pltpu.sync_copy(zb, a0.at[pl.ds(row0, RPD)])
        pltpu.sync_copy(zb, a1.at[pl.ds(row0, RPD)])
        plsc.subcore_barrier()
        base = wid * ept
        for arr, acc in zip(arrs, (a0, a1)):

            def _body(i, _):
                pltpu.sync_copy(arr.at[pl.ds(base + i * CH, CH)], ib)
                pltpu.sync_copy(ones_v, acc.at[ib], add=True)
                return 0
            lax.fori_loop(0, nch, _body, 0)
        plsc.subcore_barrier()
        for q, acc in enumerate((a0, a1)):
            pltpu.sync_copy(acc.at[pl.ds(row0, RPD)],
                            out.at[cid, 2 * p + q, pl.ds(row0, RPD)])
            @pl.when(sid == NS - 1)
            def _tail():
                pltpu.sync_copy(zb.at[pl.ds(0, NPAD - NPD)],
                                out.at[cid, 2 * p + q, pl.ds(NPD, NPAD - NPD)])


@functools.partial(
    pl.kernel,
    out_type=(jax.ShapeDtypeStruct((2, NPAD, HD), f32),
              jax.ShapeDtypeStruct((2, NPAD, HD), f32)),
    mesh=_mesh,
    scratch_types=[
        pltpu.VMEM((CH, HD), f32),
        pltpu.VMEM((CH,), i32),
        pltpu.VMEM((CH,), i32),
        pltpu.VMEM((RPD, HD), f32),
        pltpu.VMEM_SHARED((NPD, HD), f32),
        pltpu.VMEM_SHARED((NPD, HD), f32),
        pltpu.SemaphoreType.DMA,
    ],
    compiler_params=pltpu.CompilerParams(use_tc_tiling_on_sc=False),
)
def _agg_kernel(hsm, hsk, srcm, dstm, srck, dstk, outm, outk,
                rows_v, si, di, zb, accm, acck, sem):
    cid = lax.axis_index("c")
    sid = lax.axis_index("s")
    wid = cid * NS + sid
    row0 = sid * RPD

    def _zfill(r, _):
        for q in range(HD // 16):
            zb[r, pl.ds(q * 16, 16)] = jnp.zeros((16,), f32)
        return 0
    lax.fori_loop(0, RPD, _zfill, 0)
    pltpu.sync_copy(zb, accm.at[pl.ds(row0, RPD)])
    pltpu.sync_copy(zb, acck.at[pl.ds(row0, RPD)])
    plsc.subcore_barrier()

    for tab, sa, da, acc, nch, ept in (
            (hsm, srcm, dstm, accm, NCH_M, EPT_M),
            (hsk, srck, dstk, acck, NCH_K, EPT_K)):
        base = wid * ept

        def _body(i, _):
            off = base + i * CH
            pltpu.sync_copy(sa.at[pl.ds(off, CH)], si)
            pltpu.async_copy(tab.at[si], rows_v, sem).wait()
            pltpu.sync_copy(da.at[pl.ds(off, CH)], di)
            pltpu.sync_copy(rows_v, acc.at[di], add=True)
            return 0
        lax.fori_loop(0, nch, _body, 0)
    plsc.subcore_barrier()

    pltpu.sync_copy(accm.at[pl.ds(row0, RPD)], outm.at[cid, pl.ds(row0, RPD)])
    pltpu.sync_copy(acck.at[pl.ds(row0, RPD)], outk.at[cid, pl.ds(row0, RPD)])

    @pl.when(sid == NS - 1)
    def _tail():
        pltpu.sync_copy(zb.at[pl.ds(0, NPAD - NPD)],
                        outm.at[cid, pl.ds(NPD, NPAD - NPD)])
        pltpu.sync_copy(zb.at[pl.ds(0, NPAD - NPD)],
                        outk.at[cid, pl.ds(NPD, NPAD - NPD)])


def _invs_body(d_ref, iom_ref, iim_ref, iok_ref, iik_ref, swm_ref):
    d = d_ref[...]
    dm_o = (d[0, 0] + d[1, 0])[:, 0:1]
    dm_i = (d[0, 1] + d[1, 1])[:, 0:1]
    dk_o = (d[0, 2] + d[1, 2])[:, 0:1]
    dk_i = (d[0, 3] + d[1, 3])[:, 0:1]
    iom = lax.rsqrt(dm_o + 1.0)
    iim = lax.rsqrt(dm_i + 1.0)
    iok_ref[...] = jnp.where(dk_o > 0, lax.rsqrt(jnp.maximum(dk_o, 1.0)), 0.0)
    iik_ref[...] = jnp.where(dk_i > 0, lax.rsqrt(jnp.maximum(dk_i, 1.0)), 0.0)
    iom_ref[...] = iom
    iim_ref[...] = iim
    swm_ref[...] = iom * iim


def _invs(deg):
    shp = jax.ShapeDtypeStruct((NPAD, 1), f32)
    return pl.pallas_call(
        _invs_body,
        grid=(GRID,),
        in_specs=[pl.BlockSpec((2, 4, BLK, 16), lambda i: (0, 0, i, 0))],
        out_specs=[pl.BlockSpec((BLK, 1), lambda i: (i, 0))] * 5,
        out_shape=(shp,) * 5,
    )(deg)


def _kA_body(x_ref, w_ref, sr_ref, dr_ref, b_ref, db_ref, iom_ref, iok_ref,
             h_ref, hsm_ref, hsk_ref, s_ref, dk_ref):
    x = x_ref[...]
    h = jnp.dot(x, w_ref[...], preferred_element_type=f32)
    s = jax.nn.sigmoid(jnp.dot(x, sr_ref[...], preferred_element_type=f32)
                       + b_ref[...])
    dk = jnp.dot(x, dr_ref[...], preferred_element_type=f32) + db_ref[...]
    h_ref[...] = h
    hsm_ref[...] = h * iom_ref[...]
    hsk_ref[...] = h * iok_ref[...]
    s_ref[...] = s
    dk_ref[...] = dk


def _mm_specs(din):
    full = lambda shape: pl.BlockSpec(shape, lambda i: (0,) * len(shape))
    return [
        pl.BlockSpec((BLK, din), lambda i: (i, 0)),
        full((din, HD)),
        full((din, HD)),
        full((din, HD)),
        full((1, HD)),
        full((1, HD)),
        pl.BlockSpec((BLK, 1), lambda i: (i, 0)),
        pl.BlockSpec((BLK, 1), lambda i: (i, 0)),
    ]


_ROWOUT = [pl.BlockSpec((BLK, HD), lambda i: (i, 0))] * 5
_SHP5 = (jax.ShapeDtypeStruct((NPAD, HD), f32),) * 5


def _kA(x, w, sr, dr, b, db, iom, iok):
    return pl.pallas_call(
        _kA_body,
        grid=(GRID,),
        in_specs=_mm_specs(x.shape[1]),
        out_specs=_ROWOUT,
        out_shape=_SHP5,
    )(x, w, sr, dr, b, db, iom, iok)


def _combine(am_ref, ak_ref, h_ref, s_ref, dk_ref, iim_ref, iik_ref, swm_ref):
    am = am_ref[0] + am_ref[1]
    ak = ak_ref[0] + ak_ref[1]
    h = h_ref[...]
    s = s_ref[...]
    h_main = iim_ref[...] * am + swm_ref[...] * h
    tmp_knn = iik_ref[...] * ak
    return s * h_main + (1.0 - s) * tmp_knn + GAMMA * dk_ref[...] * h


def _kBA_body(am_ref, ak_ref, h_ref, s_ref, dk_ref, iim_ref, iik_ref, swm_ref,
              w_ref, sr_ref, dr_ref, b_ref, db_ref, iom_ref, iok_ref,
              h2_ref, hsm_ref, hsk_ref, s2_ref, dk2_ref):
    x2 = _combine(am_ref, ak_ref, h_ref, s_ref, dk_ref,
                  iim_ref, iik_ref, swm_ref)
    h2 = jnp.dot(x2, w_ref[...], preferred_element_type=f32)
    s2 = jax.nn.sigmoid(jnp.dot(x2, sr_ref[...], preferred_element_type=f32)
                        + b_ref[...])
    dk2 = jnp.dot(x2, dr_ref[...], preferred_element_type=f32) + db_ref[...]
    h2_ref[...] = h2
    hsm_ref[...] = h2 * iom_ref[...]
    hsk_ref[...] = h2 * iok_ref[...]
    s2_ref[...] = s2
    dk2_ref[...] = dk2


def _comb_specs():
    return [
        pl.BlockSpec((2, BLK, HD), lambda i: (0, i, 0)),
        pl.BlockSpec((2, BLK, HD), lambda i: (0, i, 0)),
        pl.BlockSpec((BLK, HD), lambda i: (i, 0)),
        pl.BlockSpec((BLK, HD), lambda i: (i, 0)),
        pl.BlockSpec((BLK, HD), lambda i: (i, 0)),
        pl.BlockSpec((BLK, 1), lambda i: (i, 0)),
        pl.BlockSpec((BLK, 1), lambda i: (i, 0)),
        pl.BlockSpec((BLK, 1), lambda i: (i, 0)),
    ]


def _kBA(am, ak, h, s, dk, iim, iik, swm, w, sr, dr, b, db, iom, iok):
    full = lambda shape: pl.BlockSpec(shape, lambda i: (0,) * len(shape))
    in_specs = _comb_specs() + [
        full((HD, HD)), full((HD, HD)), full((HD, HD)),
        full((1, HD)), full((1, HD)),
        pl.BlockSpec((BLK, 1), lambda i: (i, 0)),
        pl.BlockSpec((BLK, 1), lambda i: (i, 0)),
    ]
    return pl.pallas_call(
        _kBA_body,
        grid=(GRID,),
        in_specs=in_specs,
        out_specs=_ROWOUT,
        out_shape=_SHP5,
    )(am, ak, h, s, dk, iim, iik, swm, w, sr, dr, b, db, iom, iok)


def _kB_body(am_ref, ak_ref, h_ref, s_ref, dk_ref, iim_ref, iik_ref, swm_ref,
             out_ref):
    out_ref[...] = _combine(am_ref, ak_ref, h_ref, s_ref, dk_ref,
                            iim_ref, iik_ref, swm_ref)


def _kB(am, ak, h, s, dk, iim, iik, swm):
    return pl.pallas_call(
        _kB_body,
        grid=(GRID,),
        in_specs=_comb_specs(),
        out_specs=pl.BlockSpec((BLK, HD), lambda i: (i, 0)),
        out_shape=jax.ShapeDtypeStruct((NPAD, HD), f32),
    )(am, ak, h, s, dk, iim, iik, swm)


def _pad_edges(idx, ep):
    return jnp.concatenate([idx, jnp.full((ep - idx.shape[0],), N, i32)])


def kernel(feat, edge_index, knn_edge_index, W0, W1, scores0, scores1,
           bias0, bias1, Dk0, Dk1, Dbias0, Dbias1):
    featp = jnp.pad(feat, ((0, NPAD - N), (0, 0)))
    srcm = _pad_edges(edge_index[0], EP_M)
    dstm = _pad_edges(edge_index[1], EP_M)
    srck = _pad_edges(knn_edge_index[0], EP_K)
    dstk = _pad_edges(knn_edge_index[1], EP_K)

    deg = _deg_kernel(srcm, dstm, srck, dstk)
    iom, iim, iok, iik, swm = _invs(deg)

    sr0 = jnp.broadcast_to(scores0, (D, HD))
    dr0 = jnp.broadcast_to(Dk0, (D, HD))
    sr1 = jnp.broadcast_to(scores1, (HD, HD))
    dr1 = jnp.broadcast_to(Dk1, (HD, HD))
    b0 = jnp.broadcast_to(bias0.reshape(1, 1), (1, HD))
    db0 = jnp.broadcast_to(Dbias0.reshape(1, 1), (1, HD))
    b1 = jnp.broadcast_to(bias1.reshape(1, 1), (1, HD))
    db1 = jnp.broadcast_to(Dbias1.reshape(1, 1), (1, HD))

    h1, hs1m, hs1k, s1, dk1 = _kA(featp, W0, sr0, dr0, b0, db0, iom, iok)
    am1, ak1 = _agg_kernel(hs1m, hs1k, srcm, dstm, srck, dstk)
    h2, hs2m, hs2k, s2, dk2 = _kBA(am1, ak1, h1, s1, dk1, iim, iik, swm,
                                   W1, sr1, dr1, b1, db1, iom, iok)
    am2, ak2 = _agg_kernel(hs2m, hs2k, srcm, dstm, srck, dstk)
    x3 = _kB(am2, ak2, h2, s2, dk2, iim, iik, swm)
    return x3[:N]


# speedup vs baseline: 1.9884x; 1.1045x over previous
import functools
import jax
import jax.numpy as jnp
from jax import lax
from jax.experimental import pallas as pl
from jax.experimental.pallas import tpu as pltpu
from jax.experimental.pallas import tpu_sc as plsc

f32 = jnp.float32
i32 = jnp.int32

N = 10000
D = 128
HD = 64
GAMMA = 0.1
E = 320000
EK = 200000

NC = 2
NS = 16
NW = NC * NS
CH = 128

NPAD = 10240
NPD = 10112
RPD = NPD // NS
BLK = 1024
GRID = NPAD // BLK

NCH_M = 79
NCH_K = 49
EPT_M = NCH_M * CH
EPT_K = NCH_K * CH
EP_M = EPT_M * NW
EP_K = EPT_K * NW

_mesh = plsc.VectorSubcoreMesh(core_axis_name="c", subcore_axis_name="s")


@functools.partial(
    pl.kernel,
    out_type=jax.ShapeDtypeStruct((2, 4, NPAD, 16), f32),
    mesh=_mesh,
    scratch_types=[
        pltpu.VMEM((CH,), i32),
        pltpu.VMEM((CH, 16), f32),
        pltpu.VMEM((RPD, 16), f32),
        pltpu.VMEM_SHARED((NPD, 16), f32),
        pltpu.VMEM_SHARED((NPD, 16), f32),
    ],
    compiler_params=pltpu.CompilerParams(use_tc_tiling_on_sc=False),
)
def _deg_kernel(srcm, dstm, srck, dstk, out,
                ib, ones_v, zb, a0, a1):
    cid = lax.axis_index("c")
    sid = lax.axis_index("s")
    wid = cid * NS + sid
    row0 = sid * RPD

    def _fill(r, _):
        ones_v[r, pl.ds(0, 16)] = jnp.ones((16,), f32)
        return 0
    lax.fori_loop(0, CH, _fill, 0)

    def _zfill(r, _):
        zb[r, pl.ds(0, 16)] = jnp.zeros((16,), f32)
        return 0
    lax.fori_loop(0, RPD, _zfill, 0)

    for p, (arrs, nch, ept) in enumerate(
            (((srcm, dstm), NCH_M, EPT_M), ((srck, dstk), NCH_K, EPT_K))):
        pltpu.sync_copy(zb, a0.at[pl.ds(row0, RPD)])
        pltpu.sync_copy(zb, a1.at[pl.ds(row0, RPD)])
        plsc.subcore_barrier()
        base = wid * ept
        for arr, acc in zip(arrs, (a0, a1)):

            def _body(i, _):
                pltpu.sync_copy(arr.at[pl.ds(base + i * CH, CH)], ib)
                pltpu.sync_copy(ones_v, acc.at[ib], add=True)
                return 0
            lax.fori_loop(0, nch, _body, 0)
        plsc.subcore_barrier()
        for q, acc in enumerate((a0, a1)):
            pltpu.sync_copy(acc.at[pl.ds(row0, RPD)],
                            out.at[cid, 2 * p + q, pl.ds(row0, RPD)])
            @pl.when(sid == NS - 1)
            def _tail():
                pltpu.sync_copy(zb.at[pl.ds(0, NPAD - NPD)],
                                out.at[cid, 2 * p + q, pl.ds(NPD, NPAD - NPD)])


@functools.partial(
    pl.kernel,
    out_type=(jax.ShapeDtypeStruct((2, NPAD, HD), f32),
              jax.ShapeDtypeStruct((2, NPAD, HD), f32)),
    mesh=_mesh,
    scratch_types=[
        pltpu.VMEM((CH, HD), f32),
        pltpu.VMEM((CH,), i32),
        pltpu.VMEM((CH,), i32),
        pltpu.VMEM((RPD, HD), f32),
        pltpu.VMEM_SHARED((NPD, HD), f32),
        pltpu.VMEM_SHARED((NPD, HD), f32),
        pltpu.SemaphoreType.DMA,
    ],
    compiler_params=pltpu.CompilerParams(use_tc_tiling_on_sc=False),
)
def _agg_kernel(hsm, hsk, srcm, dstm, srck, dstk, outm, outk,
                rows_v, si, di, zb, accm, acck, sem):
    cid = lax.axis_index("c")
    sid = lax.axis_index("s")
    wid = cid * NS + sid
    row0 = sid * RPD

    def _zfill(r, _):
        for q in range(HD // 16):
            zb[r, pl.ds(q * 16, 16)] = jnp.zeros((16,), f32)
        return 0
    lax.fori_loop(0, RPD, _zfill, 0)
    pltpu.sync_copy(zb, accm.at[pl.ds(row0, RPD)])
    pltpu.sync_copy(zb, acck.at[pl.ds(row0, RPD)])
    plsc.subcore_barrier()

    for tab, sa, da, acc, nch, ept in (
            (hsm, srcm, dstm, accm, NCH_M, EPT_M),
            (hsk, srck, dstk, acck, NCH_K, EPT_K)):
        base = wid * ept

        def _body(i, _):
            off = base + i * CH
            pltpu.sync_copy(sa.at[pl.ds(off, CH)], si)
            gd = pltpu.async_copy(tab.at[si], rows_v, sem)
            pltpu.sync_copy(da.at[pl.ds(off, CH)], di)
            gd.wait()
            pltpu.sync_copy(rows_v, acc.at[di], add=True)
            return 0
        lax.fori_loop(0, nch, _body, 0)
    plsc.subcore_barrier()

    pltpu.sync_copy(accm.at[pl.ds(row0, RPD)], outm.at[cid, pl.ds(row0, RPD)])
    pltpu.sync_copy(acck.at[pl.ds(row0, RPD)], outk.at[cid, pl.ds(row0, RPD)])

    @pl.when(sid == NS - 1)
    def _tail():
        pltpu.sync_copy(zb.at[pl.ds(0, NPAD - NPD)],
                        outm.at[cid, pl.ds(NPD, NPAD - NPD)])
        pltpu.sync_copy(zb.at[pl.ds(0, NPAD - NPD)],
                        outk.at[cid, pl.ds(NPD, NPAD - NPD)])


def _invs_body(d_ref, iom_ref, iim_ref, iok_ref, iik_ref, swm_ref):
    d = d_ref[...]
    dm_o = (d[0, 0] + d[1, 0])[:, 0:1]
    dm_i = (d[0, 1] + d[1, 1])[:, 0:1]
    dk_o = (d[0, 2] + d[1, 2])[:, 0:1]
    dk_i = (d[0, 3] + d[1, 3])[:, 0:1]
    iom = lax.rsqrt(dm_o + 1.0)
    iim = lax.rsqrt(dm_i + 1.0)
    iok_ref[...] = jnp.where(dk_o > 0, lax.rsqrt(jnp.maximum(dk_o, 1.0)), 0.0)
    iik_ref[...] = jnp.where(dk_i > 0, lax.rsqrt(jnp.maximum(dk_i, 1.0)), 0.0)
    iom_ref[...] = iom
    iim_ref[...] = iim
    swm_ref[...] = iom * iim


def _invs(deg):
    shp = jax.ShapeDtypeStruct((NPAD, 1), f32)
    return pl.pallas_call(
        _invs_body,
        grid=(GRID,),
        in_specs=[pl.BlockSpec((2, 4, BLK, 16), lambda i: (0, 0, i, 0))],
        out_specs=[pl.BlockSpec((BLK, 1), lambda i: (i, 0))] * 5,
        out_shape=(shp,) * 5,
    )(deg)


def _kA_body(x_ref, w_ref, sr_ref, dr_ref, b_ref, db_ref, iom_ref, iok_ref,
             h_ref, hsm_ref, hsk_ref, s_ref, dk_ref):
    x = x_ref[...]
    h = jnp.dot(x, w_ref[...], preferred_element_type=f32)
    s = jax.nn.sigmoid(jnp.dot(x, sr_ref[...], preferred_element_type=f32)
                       + b_ref[...])
    dk = jnp.dot(x, dr_ref[...], preferred_element_type=f32) + db_ref[...]
    h_ref[...] = h
    hsm_ref[...] = h * iom_ref[...]
    hsk_ref[...] = h * iok_ref[...]
    s_ref[...] = s
    dk_ref[...] = dk


def _mm_specs(din):
    full = lambda shape: pl.BlockSpec(shape, lambda i: (0,) * len(shape))
    return [
        pl.BlockSpec((BLK, din), lambda i: (i, 0)),
        full((din, HD)),
        full((din, HD)),
        full((din, HD)),
        full((1, HD)),
        full((1, HD)),
        pl.BlockSpec((BLK, 1), lambda i: (i, 0)),
        pl.BlockSpec((BLK, 1), lambda i: (i, 0)),
    ]


_ROWOUT = [pl.BlockSpec((BLK, HD), lambda i: (i, 0))] * 5
_SHP5 = (jax.ShapeDtypeStruct((NPAD, HD), f32),) * 5


def _kA(x, w, sr, dr, b, db, iom, iok):
    return pl.pallas_call(
        _kA_body,
        grid=(GRID,),
        in_specs=_mm_specs(x.shape[1]),
        out_specs=_ROWOUT,
        out_shape=_SHP5,
    )(x, w, sr, dr, b, db, iom, iok)


def _combine(am_ref, ak_ref, h_ref, s_ref, dk_ref, iim_ref, iik_ref, swm_ref):
    am = am_ref[0] + am_ref[1]
    ak = ak_ref[0] + ak_ref[1]
    h = h_ref[...]
    s = s_ref[...]
    h_main = iim_ref[...] * am + swm_ref[...] * h
    tmp_knn = iik_ref[...] * ak
    return s * h_main + (1.0 - s) * tmp_knn + GAMMA * dk_ref[...] * h


def _kBA_body(am_ref, ak_ref, h_ref, s_ref, dk_ref, iim_ref, iik_ref, swm_ref,
              w_ref, sr_ref, dr_ref, b_ref, db_ref, iom_ref, iok_ref,
              h2_ref, hsm_ref, hsk_ref, s2_ref, dk2_ref):
    x2 = _combine(am_ref, ak_ref, h_ref, s_ref, dk_ref,
                  iim_ref, iik_ref, swm_ref)
    h2 = jnp.dot(x2, w_ref[...], preferred_element_type=f32)
    s2 = jax.nn.sigmoid(jnp.dot(x2, sr_ref[...], preferred_element_type=f32)
                        + b_ref[...])
    dk2 = jnp.dot(x2, dr_ref[...], preferred_element_type=f32) + db_ref[...]
    h2_ref[...] = h2
    hsm_ref[...] = h2 * iom_ref[...]
    hsk_ref[...] = h2 * iok_ref[...]
    s2_ref[...] = s2
    dk2_ref[...] = dk2


def _comb_specs():
    return [
        pl.BlockSpec((2, BLK, HD), lambda i: (0, i, 0)),
        pl.BlockSpec((2, BLK, HD), lambda i: (0, i, 0)),
        pl.BlockSpec((BLK, HD), lambda i: (i, 0)),
        pl.BlockSpec((BLK, HD), lambda i: (i, 0)),
        pl.BlockSpec((BLK, HD), lambda i: (i, 0)),
        pl.BlockSpec((BLK, 1), lambda i: (i, 0)),
        pl.BlockSpec((BLK, 1), lambda i: (i, 0)),
        pl.BlockSpec((BLK, 1), lambda i: (i, 0)),
    ]


def _kBA(am, ak, h, s, dk, iim, iik, swm, w, sr, dr, b, db, iom, iok):
    full = lambda shape: pl.BlockSpec(shape, lambda i: (0,) * len(shape))
    in_specs = _comb_specs() + [
        full((HD, HD)), full((HD, HD)), full((HD, HD)),
        full((1, HD)), full((1, HD)),
        pl.BlockSpec((BLK, 1), lambda i: (i, 0)),
        pl.BlockSpec((BLK, 1), lambda i: (i, 0)),
    ]
    return pl.pallas_call(
        _kBA_body,
        grid=(GRID,),
        in_specs=in_specs,
        out_specs=_ROWOUT,
        out_shape=_SHP5,
    )(am, ak, h, s, dk, iim, iik, swm, w, sr, dr, b, db, iom, iok)


def _kB_body(am_ref, ak_ref, h_ref, s_ref, dk_ref, iim_ref, iik_ref, swm_ref,
             out_ref):
    out_ref[...] = _combine(am_ref, ak_ref, h_ref, s_ref, dk_ref,
                            iim_ref, iik_ref, swm_ref)


def _kB(am, ak, h, s, dk, iim, iik, swm):
    return pl.pallas_call(
        _kB_body,
        grid=(GRID,),
        in_specs=_comb_specs(),
        out_specs=pl.BlockSpec((BLK, HD), lambda i: (i, 0)),
        out_shape=jax.ShapeDtypeStruct((NPAD, HD), f32),
    )(am, ak, h, s, dk, iim, iik, swm)


def _pad_edges(idx, ep):
    return jnp.concatenate([idx, jnp.full((ep - idx.shape[0],), N, i32)])


def kernel(feat, edge_index, knn_edge_index, W0, W1, scores0, scores1,
           bias0, bias1, Dk0, Dk1, Dbias0, Dbias1):
    featp = jnp.pad(feat, ((0, NPAD - N), (0, 0)))
    srcm = _pad_edges(edge_index[0], EP_M)
    dstm = _pad_edges(edge_index[1], EP_M)
    srck = _pad_edges(knn_edge_index[0], EP_K)
    dstk = _pad_edges(knn_edge_index[1], EP_K)

    deg = _deg_kernel(srcm, dstm, srck, dstk)
    iom, iim, iok, iik, swm = _invs(deg)

    sr0 = jnp.broadcast_to(scores0, (D, HD))
    dr0 = jnp.broadcast_to(Dk0, (D, HD))
    sr1 = jnp.broadcast_to(scores1, (HD, HD))
    dr1 = jnp.broadcast_to(Dk1, (HD, HD))
    b0 = jnp.broadcast_to(bias0.reshape(1, 1), (1, HD))
    db0 = jnp.broadcast_to(Dbias0.reshape(1, 1), (1, HD))
    b1 = jnp.broadcast_to(bias1.reshape(1, 1), (1, HD))
    db1 = jnp.broadcast_to(Dbias1.reshape(1, 1), (1, HD))

    h1, hs1m, hs1k, s1, dk1 = _kA(featp, W0, sr0, dr0, b0, db0, iom, iok)
    am1, ak1 = _agg_kernel(hs1m, hs1k, srcm, dstm, srck, dstk)
    h2, hs2m, hs2k, s2, dk2 = _kBA(am1, ak1, h1, s1, dk1, iim, iik, swm,
                                   W1, sr1, dr1, b1, db1, iom, iok)
    am2, ak2 = _agg_kernel(hs2m, hs2k, srcm, dstm, srck, dstk)
    x3 = _kB(am2, ak2, h2, s2, dk2, iim, iik, swm)
    return x3[:N]
